# baseline (speedup 1.0000x reference)
import functools
import jax
import jax.numpy as jnp
from jax import lax
from jax.experimental import pallas as pl
from jax.experimental.pallas import tpu as pltpu
from jax.experimental.pallas import tpu_sc as plsc

N = 1024
E = 65536
G = 32
F = 16
SDIM = 256
VDIM = 64
EDIM = 32
NBOND = 5
NLAYERS = 2

B = 1024
NB = E // B

_BF = jnp.bfloat16


def _silu(x):
    return x * (1.0 / (1.0 + jnp.exp(-x)))


def _dot(a, b):
    return jax.lax.dot_general(a, b, (((1,), (0,)), ((), ())),
                               preferred_element_type=jnp.float32)


def _dot_t(a, b):
    return jax.lax.dot_general(a, b, (((0,), (0,)), ((), ())),
                               preferred_element_type=jnp.float32)


def _split16(x8):
    hi = x8.astype(_BF)
    lo = (x8 - hi.astype(jnp.float32)).astype(_BF)
    return jnp.concatenate([hi, lo], axis=1)


def _iota_row(n):
    return jax.lax.broadcasted_iota(jnp.int32, (1, n), 1)


def _iota_col(n):
    return jax.lax.broadcasted_iota(jnp.int32, (n, 1), 0)


def _node_prep_kernel(x_ref, t_ref, pos_ref, bcol_ref, brow_ref,
                      wta_ref, bta_ref, wtb_ref, btb_ref,
                      wam_ref, bam_ref, watm_ref, batm_ref,
                      wbm_ref, bbm_ref, wbtm_ref, bbtm_ref,
                      ws1_ref, ws2_ref,
                      s_ref, posc_ref, posq_ref, pos16_ref, u_ref,
                      a1_ref, a2_ref):
    x = x_ref[:]
    t = t_ref[:]
    ohB = (bcol_ref[:] == _iota_row(G)).astype(jnp.float32)
    ohBT = (_iota_col(G) == brow_ref[:]).astype(jnp.float32)
    ta = t @ wta_ref[:] + bta_ref[:]
    tb = t @ wtb_ref[:] + btb_ref[:]
    s0 = x @ wam_ref[:] + bam_ref[:] + ohB @ ta
    s = s0 @ watm_ref[:] + batm_ref[:]
    pos = pos_ref[:]
    psum = ohBT @ pos
    cnt = jnp.sum(ohBT, axis=1, keepdims=True)
    mean = psum / jnp.maximum(cnt, 1.0)
    posc = pos - ohB @ mean
    s_ref[:] = s
    posc_ref[:] = posc
    xb = x @ wbm_ref[:] + bbm_ref[:]
    q = xb @ wbtm_ref[:]
    hi = posc.astype(_BF)
    lo = (posc - hi.astype(jnp.float32)).astype(_BF)
    posq_ref[:] = jnp.concatenate([hi, lo, q.astype(_BF)], axis=1)
    pos16_ref[:] = jnp.concatenate([hi, lo], axis=1)
    u_ref[:] = tb @ wbtm_ref[:] + bbtm_ref[:]
    a1_ref[:] = (s @ ws1_ref[:]).astype(_BF)
    a2_ref[:] = (s @ ws2_ref[:]).astype(_BF)


def _edge_prep_kernel(scol_ref, tcol_ref, gcol_ref,
                      posq_ref, pos16_ref, u_ref,
                      e0_ref, rnd_ref, cnt_ref):
    oh_t = (tcol_ref[:] == _iota_row(N)).astype(_BF)
    oh_s = (scol_ref[:] == _iota_row(N)).astype(_BF)
    oh_g = (gcol_ref[:] == _iota_row(G)).astype(jnp.float32)
    at = _dot(oh_t, posq_ref[:])
    asrc = _dot(oh_s, pos16_ref[:])
    pos_t = at[:, 0:8] + at[:, 8:16]
    pos_s = asrc[:, 0:8] + asrc[:, 8:16]
    e0_ref[:] = at[:, 16:48] + oh_g @ u_ref[:]
    r = pos_t - pos_s
    d2 = jnp.sum(r * r, axis=1, keepdims=True)
    d = jnp.sqrt(jnp.maximum(d2, 1e-6))
    rn = r / (1.0 + d)
    col3 = (_iota_row(8) == 3).astype(jnp.float32)
    rnd_ref[:] = rn + d * col3
    @pl.when(pl.program_id(0) == 0)
    def _():
        cnt_ref[:] = jnp.zeros_like(cnt_ref)
    cnt_ref[:] += _dot_t(oh_t, jnp.ones((B, 8), _BF))


def _layer_kernel(scol_ref, tcol_ref, e_ref, rnd_ref,
                  a1_ref, a2_ref, we_ref, wd_ref, bmsg_ref,
                  wvg_ref, weu_ref,
                  enew_ref, segm_ref, segmv_ref):
    oh_s = (scol_ref[:] == _iota_row(N)).astype(_BF)
    oh_t = (tcol_ref[:] == _iota_row(N)).astype(_BF)
    e = e_ref[:]
    rnd = rnd_ref[:]
    d = rnd[:, 3:4]
    pre = (_dot(oh_s, a1_ref[:]) + _dot(oh_t, a2_ref[:]) + e @ we_ref[:]
           + d * wd_ref[:] + bmsg_ref[:])
    m = _silu(pre)
    enew_ref[:] = e + m @ weu_ref[:]
    gate = m @ wvg_ref[:]
    mv = jnp.concatenate([rnd[:, 0:1] * gate, rnd[:, 1:2] * gate,
                          rnd[:, 2:3] * gate], axis=1)
    @pl.when(pl.program_id(0) == 0)
    def _():
        segm_ref[:] = jnp.zeros_like(segm_ref)
        segmv_ref[:] = jnp.zeros_like(segmv_ref)
    segm_ref[:] += _dot_t(oh_t, m.astype(_BF))
    segmv_ref[:] += _dot_t(oh_t, mv.astype(_BF))


def _node_update_kernel(s_ref, segm_ref, segmv_ref, cnt_ref,
                        wupd_ref, ws1_ref, ws2_ref,
                        snew_ref, vl_ref, a1_ref, a2_ref):
    cnt = jnp.maximum(cnt_ref[:, 0:1], 1.0)
    snew = s_ref[:] + (segm_ref[:] / cnt) @ wupd_ref[:]
    snew_ref[:] = snew
    vl_ref[:] = segmv_ref[:] / cnt
    a1_ref[:] = (snew @ ws1_ref[:]).astype(_BF)
    a2_ref[:] = (snew @ ws2_ref[:]).astype(_BF)


def _final_node_kernel(s_ref, v0_ref, v1_ref, posc_ref, bcol_ref, brow_ref,
                       wsm_ref, bsm_ref, w0f_ref, wcoord_ref,
                       wbond_ref, bbond_ref, b0_ref,
                       z_ref, c16_ref, wb2_ref, c0_ref):
    s2 = _silu(s_ref[:] @ wsm_ref[:] + bsm_ref[:])
    z_ref[:] = (s2 @ w0f_ref[:]).astype(_BF)
    v = v0_ref[:] + v1_ref[:]
    wc = wcoord_ref[:]
    c0c = v[:, 0:VDIM] @ wc
    c1c = v[:, VDIM:2 * VDIM] @ wc
    c2c = v[:, 2 * VDIM:3 * VDIM] @ wc
    zero5 = jnp.zeros((N, 5), jnp.float32)
    coords = posc_ref[:] + jnp.concatenate([c0c, c1c, c2c, zero5], axis=1)
    ohB = (bcol_ref[:] == _iota_row(G)).astype(jnp.float32)
    ohBT = (_iota_col(G) == brow_ref[:]).astype(jnp.float32)
    csum = ohBT @ coords
    cnt = jnp.sum(ohBT, axis=1, keepdims=True)
    mean = csum / jnp.maximum(cnt, 1.0)
    cc = coords - ohB @ mean
    hi = cc.astype(_BF)
    lo = (cc - hi.astype(jnp.float32)).astype(_BF)
    c16_ref[:] = jnp.concatenate([hi, lo], axis=1)
    wb2_ref[:] = wbond_ref[:] @ w0f_ref[:]
    c0_ref[:] = bbond_ref[:] @ w0f_ref[:] + b0_ref[:]


def _final_edge_kernel(icol_ref, jcol_ref, r1_ref, r2_ref, z_ref, c16_ref,
                       wb2_ref, c0_ref, w0d_ref, w1_ref, b1_ref, out_ref):
    oh_i = (icol_ref[:] == _iota_row(N)).astype(_BF)
    oh_j = (jcol_ref[:] == _iota_row(N)).astype(_BF)
    zp = _dot(oh_i + oh_j, z_ref[:])
    dc16 = _dot(oh_i - oh_j, c16_ref[:])
    dc = dc16[:, 0:8] + dc16[:, 8:16]
    dd = jnp.sum(dc * dc, axis=1, keepdims=True)
    esym = 0.5 * (r1_ref[:] + r2_ref[:])
    h = _silu(zp + esym @ wb2_ref[:] + dd * w0d_ref[:] + c0_ref[:])
    out_ref[:] = h @ w1_ref[:] + b1_ref[:]


_SC_NC = 2
_SC_NS = 16
_NW = _SC_NC * _SC_NS
_CH = E // _NW
_SUB = 128
_NSUB = _CH // _SUB


def _sc_scatter_ids(k1_2d, ids_2d):
    mesh = plsc.VectorSubcoreMesh(core_axis_name="c", subcore_axis_name="s")

    @functools.partial(
        pl.kernel, mesh=mesh,
        out_type=jax.ShapeDtypeStruct((N * N,), jnp.int32),
        scratch_types=[pltpu.VMEM((_NSUB, _SUB), jnp.int32),
                       pltpu.VMEM((_NSUB, _SUB), jnp.int32),
                       pltpu.SemaphoreType.DMA],
    )
    def k(k1_hbm, ids_hbm, tbl_hbm, kidx_v, vals_v, sem):
        wid = lax.axis_index("s") * _SC_NC + lax.axis_index("c")
        row0 = wid * _NSUB
        pltpu.sync_copy(k1_hbm.at[pl.ds(row0, _NSUB)], kidx_v)
        pltpu.sync_copy(ids_hbm.at[pl.ds(row0, _NSUB)], vals_v)
        copies = [pltpu.async_copy(vals_v.at[i], tbl_hbm.at[kidx_v.at[i]],
                                   sem) for i in range(_NSUB)]
        for c in copies:
            c.wait()

    return k(k1_2d, ids_2d)


def _sc_gather_sym(tbl, k1_flat, k2_flat, e2pad):
    mesh = plsc.VectorSubcoreMesh(core_axis_name="c", subcore_axis_name="s")

    @functools.partial(
        pl.kernel, mesh=mesh,
        compiler_params=pltpu.CompilerParams(use_tc_tiling_on_sc=False),
        out_type=jax.ShapeDtypeStruct((E, EDIM), jnp.float32),
        scratch_types=[pltpu.VMEM((_CH,), jnp.int32),
                       pltpu.VMEM((_CH,), jnp.int32),
                       pltpu.VMEM((_CH,), jnp.int32),
                       pltpu.VMEM((_CH, EDIM), jnp.float32),
                       pltpu.SemaphoreType.DMA],
    )
    def k(tbl_hbm, k1_hbm, k2_hbm, e2_hbm, r2_hbm,
          k2_v, w2_v, kk_v, rows_v, sem):
        wid = lax.axis_index("s") * _SC_NC + lax.axis_index("c")
        base = wid * _CH
        pltpu.sync_copy(k2_hbm.at[pl.ds(base, _CH)], k2_v)
        copies = []
        for i in range(_NSUB):
            sl = pl.ds(i * _SUB, _SUB)
            copies.append(pltpu.async_copy(tbl_hbm.at[k2_v.at[sl]],
                                           w2_v.at[sl], sem))
        for c in copies:
            c.wait()

        def _fix1(j, carry):
            s16 = pl.ds(j * 16, 16)
            w2_v[s16] = w2_v[s16] & (E - 1)
            return carry
        lax.fori_loop(0, _CH // 16, _fix1, 0)

        copies = []
        for i in range(_NSUB):
            sl = pl.ds(i * _SUB, _SUB)
            copies.append(pltpu.async_copy(k1_hbm.at[w2_v.at[sl]],
                                           kk_v.at[sl], sem))
        for c in copies:
            c.wait()

        def _fix2(j, carry):
            s16 = pl.ds(j * 16, 16)
            ok = kk_v[s16] == k2_v[s16]
            w2_v[s16] = jnp.where(ok, w2_v[s16], E)
            return carry
        lax.fori_loop(0, _CH // 16, _fix2, 0)

        copies = []
        for i in range(_NSUB):
            sl = pl.ds(i * _SUB, _SUB)
            copies.append(pltpu.async_copy(e2_hbm.at[w2_v.at[sl]],
                                           rows_v.at[pl.ds(i * _SUB, _SUB)],
                                           sem))
        for c in copies:
            c.wait()
        pltpu.sync_copy(rows_v, r2_hbm.at[pl.ds(base, _CH)])

    return k(tbl, k1_flat, k2_flat, e2pad)


def _row(v):
    return v.reshape(1, -1)


def kernel(x, t, pos, edge_index_local, edge_index_global, batch,
           batch_edge_global, params):
    p = params
    src = edge_index_global[0].astype(jnp.int32)
    tgt = edge_index_global[1].astype(jnp.int32)
    beg = batch_edge_global.astype(jnp.int32)
    batch = batch.astype(jnp.int32)
    pos8 = jnp.pad(pos, ((0, 0), (0, 5)))

    scol = src.reshape(E, 1)
    tcol = tgt.reshape(E, 1)
    gcol = beg.reshape(E, 1)
    bcol = batch.reshape(N, 1)
    brow = batch.reshape(1, N)

    wmsg0, wmsg1 = p['Wmsg0'], p['Wmsg1']
    ws1_0, ws2_0 = wmsg0[:SDIM], wmsg0[SDIM:2 * SDIM]
    we_0, wd_0 = wmsg0[2 * SDIM:2 * SDIM + EDIM], _row(wmsg0[2 * SDIM + EDIM])
    ws1_1, ws2_1 = wmsg1[:SDIM], wmsg1[SDIM:2 * SDIM]
    we_1, wd_1 = wmsg1[2 * SDIM:2 * SDIM + EDIM], _row(wmsg1[2 * SDIM + EDIM])
    w0f, w0d = p['W0'][:SDIM], _row(p['W0'][SDIM])
    w1p = jnp.pad(p['W1'], ((0, 0), (0, 3)))
    b1p = _row(jnp.pad(p['b1'], (0, 3)))

    f32 = jnp.float32
    full = lambda shape: pl.BlockSpec(shape, lambda i: (0,) * len(shape))
    ecol = pl.BlockSpec((B, 1), lambda i: (i, 0))
    eblk = lambda w: pl.BlockSpec((B, w), lambda i: (i, 0))

    s, posc, posq, pos16, u, a1, a2 = pl.pallas_call(
        _node_prep_kernel,
        out_shape=[jax.ShapeDtypeStruct((N, SDIM), f32),
                   jax.ShapeDtypeStruct((N, 8), f32),
                   jax.ShapeDtypeStruct((N, 48), _BF),
                   jax.ShapeDtypeStruct((N, 16), _BF),
                   jax.ShapeDtypeStruct((G, EDIM), f32),
                   jax.ShapeDtypeStruct((N, SDIM), _BF),
                   jax.ShapeDtypeStruct((N, SDIM), _BF)],
    )(x, t, pos8, bcol, brow, p['Wta'], _row(p['bta']), p['Wtb'],
      _row(p['btb']), p['Wam'], _row(p['bam']), p['Watm'], _row(p['batm']),
      p['Wbm'], _row(p['bbm']), p['Wbtm'], _row(p['bbtm']), ws1_0, ws2_0)

    tlhs = pltpu.CompilerParams(fuse_transposed_lhs_in_matmul=True)
    e0, rnd, cnt8 = pl.pallas_call(
        _edge_prep_kernel,
        grid=(NB,),
        in_specs=[ecol, ecol, ecol, full((N, 48)), full((N, 16)),
                  full((G, EDIM))],
        out_specs=[eblk(EDIM), eblk(8), full((N, 8))],
        out_shape=[jax.ShapeDtypeStruct((E, EDIM), f32),
                   jax.ShapeDtypeStruct((E, 8), f32),
                   jax.ShapeDtypeStruct((N, 8), f32)],
        compiler_params=tlhs,
    )(scol, tcol, gcol, posq, pos16, u)

    layer_call = pl.pallas_call(
        _layer_kernel,
        grid=(NB,),
        in_specs=[ecol, ecol, eblk(EDIM), eblk(8),
                  full((N, SDIM)), full((N, SDIM)), full((EDIM, SDIM)),
                  full((1, SDIM)), full((1, SDIM)), full((SDIM, VDIM)),
                  full((SDIM, EDIM))],
        out_specs=[eblk(EDIM), full((N, SDIM)), full((N, 3 * VDIM))],
        out_shape=[jax.ShapeDtypeStruct((E, EDIM), f32),
                   jax.ShapeDtypeStruct((N, SDIM), f32),
                   jax.ShapeDtypeStruct((N, 3 * VDIM), f32)],
        compiler_params=tlhs,
    )
    node_update = pl.pallas_call(
        _node_update_kernel,
        out_shape=[jax.ShapeDtypeStruct((N, SDIM), f32),
                   jax.ShapeDtypeStruct((N, 3 * VDIM), f32),
                   jax.ShapeDtypeStruct((N, SDIM), _BF),
                   jax.ShapeDtypeStruct((N, SDIM), _BF)],
    )

    e1, segm0, segmv0 = layer_call(scol, tcol, e0, rnd, a1, a2,
                                   we_0, wd_0, _row(p['bmsg0']),
                                   p['Wvg0'], p['Weu0'])
    s1, v0, a1b, a2b = node_update(s, segm0, segmv0, cnt8,
                                   p['Wupd0'], ws1_1, ws2_1)
    e2, segm1, segmv1 = layer_call(scol, tcol, e1, rnd, a1b, a2b,
                                   we_1, wd_1, _row(p['bmsg1']),
                                   p['Wvg1'], p['Weu1'])
    s2f, v1, _, _ = node_update(s1, segm1, segmv1, cnt8,
                                p['Wupd1'], ws1_1, ws2_1)

    z, c16, wb2, c0v = pl.pallas_call(
        _final_node_kernel,
        out_shape=[jax.ShapeDtypeStruct((N, SDIM), _BF),
                   jax.ShapeDtypeStruct((N, 16), _BF),
                   jax.ShapeDtypeStruct((EDIM, SDIM), f32),
                   jax.ShapeDtypeStruct((1, SDIM), f32)],
    )(s2f, v0, v1, posc, bcol, brow, p['Wsm'], _row(p['bsm']), w0f,
      p['Wcoord'], p['Wbond'], _row(p['bbond']), _row(p['b0']))

    key1 = src * N + tgt
    key2 = tgt * N + src
    ids = jnp.arange(E, dtype=jnp.int32)
    tbl = _sc_scatter_ids(key1.reshape(E // _SUB, _SUB),
                          ids.reshape(E // _SUB, _SUB))
    e2pad = jnp.concatenate([e2, jnp.zeros((8, EDIM), e2.dtype)], axis=0)
    r2 = _sc_gather_sym(tbl, key1, key2, e2pad)

    outp = pl.pallas_call(
        _final_edge_kernel,
        grid=(NB,),
        in_specs=[ecol, ecol, eblk(EDIM), eblk(EDIM), full((N, SDIM)),
                  full((N, 16)), full((EDIM, SDIM)), full((1, SDIM)),
                  full((1, SDIM)), full((SDIM, 8)), full((1, 8))],
        out_specs=eblk(8),
        out_shape=jax.ShapeDtypeStruct((E, 8), f32),
    )(tcol, scol, e2, r2, z, c16, wb2, c0v, w0d, w1p, b1p)

    return outp[:, :NBOND]


# speedup vs baseline: 1.2215x; 1.2215x over previous
import functools
import jax
import jax.numpy as jnp
from jax import lax
from jax.experimental import pallas as pl
from jax.experimental.pallas import tpu as pltpu
from jax.experimental.pallas import tpu_sc as plsc

N = 1024
E = 65536
G = 32
F = 16
SDIM = 256
VDIM = 64
EDIM = 32
NBOND = 5
NLAYERS = 2

B = 1024
NB = E // B

_BF = jnp.bfloat16


def _silu(x):
    return x * (1.0 / (1.0 + jnp.exp(-x)))


def _dot(a, b):
    return jax.lax.dot_general(a, b, (((1,), (0,)), ((), ())),
                               preferred_element_type=jnp.float32)


def _dot_t(a, b):
    return jax.lax.dot_general(a, b, (((0,), (0,)), ((), ())),
                               preferred_element_type=jnp.float32)


def _split16(x8):
    hi = x8.astype(_BF)
    lo = (x8 - hi.astype(jnp.float32)).astype(_BF)
    return jnp.concatenate([hi, lo], axis=1)


def _iota_row(n):
    return jax.lax.broadcasted_iota(jnp.int32, (1, n), 1)


def _iota_col(n):
    return jax.lax.broadcasted_iota(jnp.int32, (n, 1), 0)


def _node_prep_kernel(x_ref, t_ref, pos_ref, bcol_ref, brow_ref,
                      wta_ref, bta_ref, wtb_ref, btb_ref,
                      wam_ref, bam_ref, watm_ref, batm_ref,
                      wbm_ref, bbm_ref, wbtm_ref, bbtm_ref,
                      ws1_ref, ws2_ref,
                      s_ref, posc_ref, ta1_ref, ta2_ref, u_ref):
    x = x_ref[:]
    t = t_ref[:]
    ohB = (bcol_ref[:] == _iota_row(G)).astype(jnp.float32)
    ohBT = (_iota_col(G) == brow_ref[:]).astype(jnp.float32)
    ta = t @ wta_ref[:] + bta_ref[:]
    tb = t @ wtb_ref[:] + btb_ref[:]
    s0 = x @ wam_ref[:] + bam_ref[:] + ohB @ ta
    s = s0 @ watm_ref[:] + batm_ref[:]
    pos = pos_ref[:]
    psum = ohBT @ pos
    cnt = jnp.sum(ohBT, axis=1, keepdims=True)
    mean = psum / jnp.maximum(cnt, 1.0)
    posc = pos - ohB @ mean
    s_ref[:] = s
    posc_ref[:] = posc
    xb = x @ wbm_ref[:] + bbm_ref[:]
    q = xb @ wbtm_ref[:]
    hi = posc.astype(_BF)
    lo = (posc - hi.astype(jnp.float32)).astype(_BF)
    u_ref[:] = tb @ wbtm_ref[:] + bbtm_ref[:]
    a1 = (s @ ws1_ref[:]).astype(_BF)
    a2 = (s @ ws2_ref[:]).astype(_BF)
    ta1_ref[:] = jnp.concatenate([a1, hi, lo], axis=1)
    ta2_ref[:] = jnp.concatenate([a2, hi, lo, q.astype(_BF)], axis=1)


def _layer0_kernel(scol_ref, tcol_ref, gcol_ref,
                   ta2_ref, ta1_ref, u_ref,
                   we_ref, wd_ref, bmsg_ref, wvg_ref, weu_ref,
                   e1_ref, rnd_ref, cnt_ref, segm_ref, segmv_ref):
    @pl.when(pl.program_id(0) == 0)
    def _():
        cnt_ref[:] = jnp.zeros_like(cnt_ref)
        segm_ref[:] = jnp.zeros_like(segm_ref)
        segmv_ref[:] = jnp.zeros_like(segmv_ref)
    ta2 = ta2_ref[:]
    ta1 = ta1_ref[:]
    u = u_ref[:]
    we = we_ref[:]
    wd = wd_ref[:]
    bmsg = bmsg_ref[:]
    wvg = wvg_ref[:]
    weu = weu_ref[:]
    col3 = (_iota_row(8) == 3).astype(jnp.float32)
    H = B // _NSPLIT
    for h in range(_NSPLIT):
        sl = pl.ds(h * H, H)
        oh_t = (tcol_ref[sl, :] == _iota_row(N)).astype(_BF)
        oh_s = (scol_ref[sl, :] == _iota_row(N)).astype(_BF)
        oh_g = (gcol_ref[sl, :] == _iota_row(G)).astype(jnp.float32)
        at = _dot(oh_t, ta2)
        asrc = _dot(oh_s, ta1)
        pos_t = at[:, SDIM:SDIM + 8] + at[:, SDIM + 8:SDIM + 16]
        pos_s = asrc[:, SDIM:SDIM + 8] + asrc[:, SDIM + 8:SDIM + 16]
        e0 = at[:, SDIM + 16:SDIM + 48] + oh_g @ u
        r = pos_t - pos_s
        d2 = jnp.sum(r * r, axis=1, keepdims=True)
        d = jnp.sqrt(jnp.maximum(d2, 1e-6))
        rn = r / (1.0 + d)
        rnd_ref[sl, :] = rn + d * col3
        pre = (asrc[:, 0:SDIM] + at[:, 0:SDIM] + e0 @ we
               + d * wd + bmsg)
        m = _silu(pre)
        e1_ref[sl, :] = e0 + m @ weu
        gate = m @ wvg
        mv = jnp.concatenate([rn[:, 0:1] * gate, rn[:, 1:2] * gate,
                              rn[:, 2:3] * gate], axis=1)
        cnt_ref[:] += _dot_t(oh_t, jnp.ones((H, 8), _BF))
        segm_ref[:] += _dot_t(oh_t, m.astype(_BF))
        segmv_ref[:] += _dot_t(oh_t, mv.astype(_BF))


_NSPLIT = 2


def _layer_kernel(scol_ref, tcol_ref, e_ref, rnd_ref,
                  a1_ref, a2_ref, we_ref, wd_ref, bmsg_ref,
                  wvg_ref, weu_ref,
                  enew_ref, segm_ref, segmv_ref):
    @pl.when(pl.program_id(0) == 0)
    def _():
        segm_ref[:] = jnp.zeros_like(segm_ref)
        segmv_ref[:] = jnp.zeros_like(segmv_ref)
    a1 = a1_ref[:]
    a2 = a2_ref[:]
    we = we_ref[:]
    wd = wd_ref[:]
    bmsg = bmsg_ref[:]
    wvg = wvg_ref[:]
    weu = weu_ref[:]
    H = B // _NSPLIT
    for h in range(_NSPLIT):
        sl = pl.ds(h * H, H)
        oh_s = (scol_ref[sl, :] == _iota_row(N)).astype(_BF)
        oh_t = (tcol_ref[sl, :] == _iota_row(N)).astype(_BF)
        e = e_ref[sl, :]
        rnd = rnd_ref[sl, :]
        d = rnd[:, 3:4]
        pre = (_dot(oh_s, a1) + _dot(oh_t, a2) + e @ we + d * wd + bmsg)
        m = _silu(pre)
        enew_ref[sl, :] = e + m @ weu
        gate = m @ wvg
        mv = jnp.concatenate([rnd[:, 0:1] * gate, rnd[:, 1:2] * gate,
                              rnd[:, 2:3] * gate], axis=1)
        segm_ref[:] += _dot_t(oh_t, m.astype(_BF))
        segmv_ref[:] += _dot_t(oh_t, mv.astype(_BF))


def _node_update_kernel(s_ref, segm_ref, segmv_ref, cnt_ref,
                        wupd_ref, ws1_ref, ws2_ref,
                        snew_ref, vl_ref, a1_ref, a2_ref):
    cnt = jnp.maximum(cnt_ref[:, 0:1], 1.0)
    snew = s_ref[:] + (segm_ref[:] / cnt) @ wupd_ref[:]
    snew_ref[:] = snew
    vl_ref[:] = segmv_ref[:] / cnt
    a1_ref[:] = (snew @ ws1_ref[:]).astype(_BF)
    a2_ref[:] = (snew @ ws2_ref[:]).astype(_BF)


def _final_node_kernel(s_ref, v0_ref, v1_ref, posc_ref, bcol_ref, brow_ref,
                       wsm_ref, bsm_ref, w0f_ref, wcoord_ref,
                       wbond_ref, bbond_ref, b0_ref,
                       z_ref, c16_ref, wb2_ref, c0_ref):
    s2 = _silu(s_ref[:] @ wsm_ref[:] + bsm_ref[:])
    z_ref[:] = (s2 @ w0f_ref[:]).astype(_BF)
    v = v0_ref[:] + v1_ref[:]
    wc = wcoord_ref[:]
    c0c = v[:, 0:VDIM] @ wc
    c1c = v[:, VDIM:2 * VDIM] @ wc
    c2c = v[:, 2 * VDIM:3 * VDIM] @ wc
    zero5 = jnp.zeros((N, 5), jnp.float32)
    coords = posc_ref[:] + jnp.concatenate([c0c, c1c, c2c, zero5], axis=1)
    ohB = (bcol_ref[:] == _iota_row(G)).astype(jnp.float32)
    ohBT = (_iota_col(G) == brow_ref[:]).astype(jnp.float32)
    csum = ohBT @ coords
    cnt = jnp.sum(ohBT, axis=1, keepdims=True)
    mean = csum / jnp.maximum(cnt, 1.0)
    cc = coords - ohB @ mean
    hi = cc.astype(_BF)
    lo = (cc - hi.astype(jnp.float32)).astype(_BF)
    c16_ref[:] = jnp.concatenate([hi, lo], axis=1)
    wb2_ref[:] = wbond_ref[:] @ w0f_ref[:]
    c0_ref[:] = bbond_ref[:] @ w0f_ref[:] + b0_ref[:]


def _final_edge_kernel(icol_ref, jcol_ref, r1_ref, r2_ref, z_ref, c16_ref,
                       wb2_ref, c0_ref, w0d_ref, w1_ref, b1_ref, out_ref):
    oh_i = (icol_ref[:] == _iota_row(N)).astype(_BF)
    oh_j = (jcol_ref[:] == _iota_row(N)).astype(_BF)
    zp = _dot(oh_i + oh_j, z_ref[:])
    dc16 = _dot(oh_i - oh_j, c16_ref[:])
    dc = dc16[:, 0:8] + dc16[:, 8:16]
    dd = jnp.sum(dc * dc, axis=1, keepdims=True)
    esym = 0.5 * (r1_ref[:] + r2_ref[:])
    h = _silu(zp + esym @ wb2_ref[:] + dd * w0d_ref[:] + c0_ref[:])
    out_ref[:] = h @ w1_ref[:] + b1_ref[:]


_SC_NC = 2
_SC_NS = 16
_NW = _SC_NC * _SC_NS
_CH = E // _NW
_SUB = 128
_NSUB = _CH // _SUB


def _sc_scatter_ids(k1_2d, ids_2d):
    mesh = plsc.VectorSubcoreMesh(core_axis_name="c", subcore_axis_name="s")

    @functools.partial(
        pl.kernel, mesh=mesh,
        out_type=jax.ShapeDtypeStruct((N * N,), jnp.int32),
        scratch_types=[pltpu.VMEM((_NSUB, _SUB), jnp.int32),
                       pltpu.VMEM((_NSUB, _SUB), jnp.int32),
                       pltpu.SemaphoreType.DMA],
    )
    def k(k1_hbm, ids_hbm, tbl_hbm, kidx_v, vals_v, sem):
        wid = lax.axis_index("s") * _SC_NC + lax.axis_index("c")
        row0 = wid * _NSUB
        pltpu.sync_copy(k1_hbm.at[pl.ds(row0, _NSUB)], kidx_v)
        pltpu.sync_copy(ids_hbm.at[pl.ds(row0, _NSUB)], vals_v)
        copies = [pltpu.async_copy(vals_v.at[i], tbl_hbm.at[kidx_v.at[i]],
                                   sem) for i in range(_NSUB)]
        for c in copies:
            c.wait()

    return k(k1_2d, ids_2d)


def _sc_gather_sym(tbl, k1_flat, k2_flat, e2pad):
    mesh = plsc.VectorSubcoreMesh(core_axis_name="c", subcore_axis_name="s")

    @functools.partial(
        pl.kernel, mesh=mesh,
        compiler_params=pltpu.CompilerParams(use_tc_tiling_on_sc=False),
        out_type=jax.ShapeDtypeStruct((E, EDIM), jnp.float32),
        scratch_types=[pltpu.VMEM((_CH,), jnp.int32),
                       pltpu.VMEM((_CH,), jnp.int32),
                       pltpu.VMEM((_CH,), jnp.int32),
                       pltpu.VMEM((_CH, EDIM), jnp.float32),
                       pltpu.SemaphoreType.DMA],
    )
    def k(tbl_hbm, k1_hbm, k2_hbm, e2_hbm, r2_hbm,
          k2_v, w2_v, kk_v, rows_v, sem):
        wid = lax.axis_index("s") * _SC_NC + lax.axis_index("c")
        base = wid * _CH
        pltpu.sync_copy(k2_hbm.at[pl.ds(base, _CH)], k2_v)
        copies = []
        for i in range(_NSUB):
            sl = pl.ds(i * _SUB, _SUB)
            copies.append(pltpu.async_copy(tbl_hbm.at[k2_v.at[sl]],
                                           w2_v.at[sl], sem))
        for c in copies:
            c.wait()

        def _fix1(j, carry):
            s16 = pl.ds(j * 16, 16)
            w2_v[s16] = w2_v[s16] & (E - 1)
            return carry
        lax.fori_loop(0, _CH // 16, _fix1, 0)

        copies = []
        for i in range(_NSUB):
            sl = pl.ds(i * _SUB, _SUB)
            copies.append(pltpu.async_copy(k1_hbm.at[w2_v.at[sl]],
                                           kk_v.at[sl], sem))
        for c in copies:
            c.wait()

        def _fix2(j, carry):
            s16 = pl.ds(j * 16, 16)
            ok = kk_v[s16] == k2_v[s16]
            w2_v[s16] = jnp.where(ok, w2_v[s16], E)
            return carry
        lax.fori_loop(0, _CH // 16, _fix2, 0)

        copies = []
        for i in range(_NSUB):
            sl = pl.ds(i * _SUB, _SUB)
            copies.append(pltpu.async_copy(e2_hbm.at[w2_v.at[sl]],
                                           rows_v.at[pl.ds(i * _SUB, _SUB)],
                                           sem))
        for c in copies:
            c.wait()
        pltpu.sync_copy(rows_v, r2_hbm.at[pl.ds(base, _CH)])

    return k(tbl, k1_flat, k2_flat, e2pad)


def _row(v):
    return v.reshape(1, -1)


def kernel(x, t, pos, edge_index_local, edge_index_global, batch,
           batch_edge_global, params):
    p = params
    src = edge_index_global[0].astype(jnp.int32)
    tgt = edge_index_global[1].astype(jnp.int32)
    beg = batch_edge_global.astype(jnp.int32)
    batch = batch.astype(jnp.int32)
    pos8 = jnp.pad(pos, ((0, 0), (0, 5)))

    scol = src.reshape(E, 1)
    tcol = tgt.reshape(E, 1)
    gcol = beg.reshape(E, 1)
    bcol = batch.reshape(N, 1)
    brow = batch.reshape(1, N)

    wmsg0, wmsg1 = p['Wmsg0'], p['Wmsg1']
    ws1_0, ws2_0 = wmsg0[:SDIM], wmsg0[SDIM:2 * SDIM]
    we_0, wd_0 = wmsg0[2 * SDIM:2 * SDIM + EDIM], _row(wmsg0[2 * SDIM + EDIM])
    ws1_1, ws2_1 = wmsg1[:SDIM], wmsg1[SDIM:2 * SDIM]
    we_1, wd_1 = wmsg1[2 * SDIM:2 * SDIM + EDIM], _row(wmsg1[2 * SDIM + EDIM])
    w0f, w0d = p['W0'][:SDIM], _row(p['W0'][SDIM])
    w1p = jnp.pad(p['W1'], ((0, 0), (0, 3)))
    b1p = _row(jnp.pad(p['b1'], (0, 3)))

    f32 = jnp.float32
    full = lambda shape: pl.BlockSpec(shape, lambda i: (0,) * len(shape))
    ecol = pl.BlockSpec((B, 1), lambda i: (i, 0))
    eblk = lambda w: pl.BlockSpec((B, w), lambda i: (i, 0))

    s, posc, ta1, ta2, u = pl.pallas_call(
        _node_prep_kernel,
        out_shape=[jax.ShapeDtypeStruct((N, SDIM), f32),
                   jax.ShapeDtypeStruct((N, 8), f32),
                   jax.ShapeDtypeStruct((N, SDIM + 16), _BF),
                   jax.ShapeDtypeStruct((N, SDIM + 48), _BF),
                   jax.ShapeDtypeStruct((G, EDIM), f32)],
    )(x, t, pos8, bcol, brow, p['Wta'], _row(p['bta']), p['Wtb'],
      _row(p['btb']), p['Wam'], _row(p['bam']), p['Watm'], _row(p['batm']),
      p['Wbm'], _row(p['bbm']), p['Wbtm'], _row(p['bbtm']), ws1_0, ws2_0)

    tlhs = pltpu.CompilerParams(fuse_transposed_lhs_in_matmul=True)
    e1, rnd, cnt8, segm0, segmv0 = pl.pallas_call(
        _layer0_kernel,
        grid=(NB,),
        in_specs=[ecol, ecol, ecol, full((N, SDIM + 48)),
                  full((N, SDIM + 16)), full((G, EDIM)),
                  full((EDIM, SDIM)), full((1, SDIM)), full((1, SDIM)),
                  full((SDIM, VDIM)), full((SDIM, EDIM))],
        out_specs=[eblk(EDIM), eblk(8), full((N, 8)), full((N, SDIM)),
                   full((N, 3 * VDIM))],
        out_shape=[jax.ShapeDtypeStruct((E, EDIM), f32),
                   jax.ShapeDtypeStruct((E, 8), f32),
                   jax.ShapeDtypeStruct((N, 8), f32),
                   jax.ShapeDtypeStruct((N, SDIM), f32),
                   jax.ShapeDtypeStruct((N, 3 * VDIM), f32)],
        compiler_params=tlhs,
    )(scol, tcol, gcol, ta2, ta1, u, we_0, wd_0, _row(p['bmsg0']),
      p['Wvg0'], p['Weu0'])

    layer_call = pl.pallas_call(
        _layer_kernel,
        grid=(NB,),
        in_specs=[ecol, ecol, eblk(EDIM), eblk(8),
                  full((N, SDIM)), full((N, SDIM)), full((EDIM, SDIM)),
                  full((1, SDIM)), full((1, SDIM)), full((SDIM, VDIM)),
                  full((SDIM, EDIM))],
        out_specs=[eblk(EDIM), full((N, SDIM)), full((N, 3 * VDIM))],
        out_shape=[jax.ShapeDtypeStruct((E, EDIM), f32),
                   jax.ShapeDtypeStruct((N, SDIM), f32),
                   jax.ShapeDtypeStruct((N, 3 * VDIM), f32)],
        compiler_params=tlhs,
    )
    node_update = pl.pallas_call(
        _node_update_kernel,
        out_shape=[jax.ShapeDtypeStruct((N, SDIM), f32),
                   jax.ShapeDtypeStruct((N, 3 * VDIM), f32),
                   jax.ShapeDtypeStruct((N, SDIM), _BF),
                   jax.ShapeDtypeStruct((N, SDIM), _BF)],
    )

    s1, v0, a1b, a2b = node_update(s, segm0, segmv0, cnt8,
                                   p['Wupd0'], ws1_1, ws2_1)
    e2, segm1, segmv1 = layer_call(scol, tcol, e1, rnd, a1b, a2b,
                                   we_1, wd_1, _row(p['bmsg1']),
                                   p['Wvg1'], p['Weu1'])
    s2f, v1, _, _ = node_update(s1, segm1, segmv1, cnt8,
                                p['Wupd1'], ws1_1, ws2_1)

    z, c16, wb2, c0v = pl.pallas_call(
        _final_node_kernel,
        out_shape=[jax.ShapeDtypeStruct((N, SDIM), _BF),
                   jax.ShapeDtypeStruct((N, 16), _BF),
                   jax.ShapeDtypeStruct((EDIM, SDIM), f32),
                   jax.ShapeDtypeStruct((1, SDIM), f32)],
    )(s2f, v0, v1, posc, bcol, brow, p['Wsm'], _row(p['bsm']), w0f,
      p['Wcoord'], p['Wbond'], _row(p['bbond']), _row(p['b0']))

    key1 = src * N + tgt
    key2 = tgt * N + src
    ids = jnp.arange(E, dtype=jnp.int32)
    tbl = _sc_scatter_ids(key1.reshape(E // _SUB, _SUB),
                          ids.reshape(E // _SUB, _SUB))
    e2pad = jnp.concatenate([e2, jnp.zeros((8, EDIM), e2.dtype)], axis=0)
    r2 = _sc_gather_sym(tbl, key1, key2, e2pad)

    outp = pl.pallas_call(
        _final_edge_kernel,
        grid=(NB,),
        in_specs=[ecol, ecol, eblk(EDIM), eblk(EDIM), full((N, SDIM)),
                  full((N, 16)), full((EDIM, SDIM)), full((1, SDIM)),
                  full((1, SDIM)), full((SDIM, 8)), full((1, 8))],
        out_specs=eblk(8),
        out_shape=jax.ShapeDtypeStruct((E, 8), f32),
    )(tcol, scol, e2, r2, z, c16, wb2, c0v, w0d, w1p, b1p)

    return outp[:, :NBOND]


# speedup vs baseline: 1.6589x; 1.3581x over previous
import functools
import jax
import jax.numpy as jnp
from jax import lax
from jax.experimental import pallas as pl
from jax.experimental.pallas import tpu as pltpu
from jax.experimental.pallas import tpu_sc as plsc

N = 1024
E = 65536
G = 32
F = 16
SDIM = 256
VDIM = 64
EDIM = 32
NBOND = 5
NLAYERS = 2

B = 1024
NB = E // B

_BF = jnp.bfloat16


def _silu(x):
    return x * (1.0 / (1.0 + jnp.exp(-x)))


def _dot(a, b):
    return jax.lax.dot_general(a, b, (((1,), (0,)), ((), ())),
                               preferred_element_type=jnp.float32)


def _dot_t(a, b):
    return jax.lax.dot_general(a, b, (((0,), (0,)), ((), ())),
                               preferred_element_type=jnp.float32)


def _split16(x8):
    hi = x8.astype(_BF)
    lo = (x8 - hi.astype(jnp.float32)).astype(_BF)
    return jnp.concatenate([hi, lo], axis=1)


def _iota_row(n):
    return jax.lax.broadcasted_iota(jnp.int32, (1, n), 1)


def _iota_col(n):
    return jax.lax.broadcasted_iota(jnp.int32, (n, 1), 0)


def _node_prep_kernel(x_ref, t_ref, pos_ref, bcol_ref, brow_ref,
                      wta_ref, bta_ref, wtb_ref, btb_ref,
                      wam_ref, bam_ref, watm_ref, batm_ref,
                      wbm_ref, bbm_ref, wbtm_ref, bbtm_ref,
                      ws1_ref, ws2_ref,
                      s_ref, posc_ref, ta1_ref, ta2_ref, u_ref):
    x = x_ref[:]
    t = t_ref[:]
    ohB = (bcol_ref[:] == _iota_row(G)).astype(jnp.float32)
    ohBT = (_iota_col(G) == brow_ref[:]).astype(jnp.float32)
    ta = t @ wta_ref[:] + bta_ref[:]
    tb = t @ wtb_ref[:] + btb_ref[:]
    s0 = x @ wam_ref[:] + bam_ref[:] + ohB @ ta
    s = s0 @ watm_ref[:] + batm_ref[:]
    pos = pos_ref[:]
    psum = ohBT @ pos
    cnt = jnp.sum(ohBT, axis=1, keepdims=True)
    mean = psum / jnp.maximum(cnt, 1.0)
    posc = pos - ohB @ mean
    s_ref[:] = s
    posc_ref[:] = posc
    xb = x @ wbm_ref[:] + bbm_ref[:]
    q = xb @ wbtm_ref[:]
    hi = posc.astype(_BF)
    lo = (posc - hi.astype(jnp.float32)).astype(_BF)
    u_ref[:] = tb @ wbtm_ref[:] + bbtm_ref[:]
    a1 = (s @ ws1_ref[:]).astype(_BF)
    a2 = (s @ ws2_ref[:]).astype(_BF)
    ta1_ref[:] = jnp.concatenate([a1, hi, lo], axis=1)
    ta2_ref[:] = jnp.concatenate([a2, hi, lo, q.astype(_BF)], axis=1)


def _layer0_kernel(scol_ref, tcol_ref, gcol_ref,
                   ta2_ref, ta1_ref, u_ref,
                   we_ref, wd_ref, bmsg_ref, wvg_ref, weu_ref,
                   e1_ref, rnd_ref, cnt_ref, segm_ref, segmv_ref):
    @pl.when(pl.program_id(0) == 0)
    def _():
        cnt_ref[:] = jnp.zeros_like(cnt_ref)
        segm_ref[:] = jnp.zeros_like(segm_ref)
        segmv_ref[:] = jnp.zeros_like(segmv_ref)
    ta2 = ta2_ref[:]
    ta1 = ta1_ref[:]
    u = u_ref[:]
    we = we_ref[:]
    wd = wd_ref[:]
    bmsg = bmsg_ref[:]
    wvg = wvg_ref[:]
    weu = weu_ref[:]
    col3 = (_iota_row(8) == 3).astype(jnp.float32)
    H = B // _NSPLIT
    for h in range(_NSPLIT):
        sl = pl.ds(h * H, H)
        oh_t = (tcol_ref[sl, :] == _iota_row(N)).astype(_BF)
        oh_s = (scol_ref[sl, :] == _iota_row(N)).astype(_BF)
        oh_g = (gcol_ref[sl, :] == _iota_row(G)).astype(jnp.float32)
        at = _dot(oh_t, ta2)
        asrc = _dot(oh_s, ta1)
        pos_t = at[:, SDIM:SDIM + 8] + at[:, SDIM + 8:SDIM + 16]
        pos_s = asrc[:, SDIM:SDIM + 8] + asrc[:, SDIM + 8:SDIM + 16]
        e0 = at[:, SDIM + 16:SDIM + 48] + oh_g @ u
        r = pos_t - pos_s
        d2 = jnp.sum(r * r, axis=1, keepdims=True)
        d = jnp.sqrt(jnp.maximum(d2, 1e-6))
        rn = r / (1.0 + d)
        rnd_ref[sl, :] = rn + d * col3
        pre = (asrc[:, 0:SDIM] + at[:, 0:SDIM] + e0 @ we
               + d * wd + bmsg)
        m = _silu(pre)
        e1_ref[sl, :] = e0 + m @ weu
        gate = m @ wvg
        mv = jnp.concatenate([rn[:, 0:1] * gate, rn[:, 1:2] * gate,
                              rn[:, 2:3] * gate], axis=1)
        cnt_ref[:] += _dot_t(oh_t, jnp.ones((H, 8), _BF))
        segm_ref[:] += _dot_t(oh_t, m.astype(_BF))
        segmv_ref[:] += _dot_t(oh_t, mv.astype(_BF))


_NSPLIT = 2


def _layer_kernel(scol_ref, tcol_ref, e_ref, rnd_ref,
                  a1_ref, a2_ref, we_ref, wd_ref, bmsg_ref,
                  wvg_ref, weu_ref,
                  enew_ref, segm_ref, segmv_ref):
    @pl.when(pl.program_id(0) == 0)
    def _():
        segm_ref[:] = jnp.zeros_like(segm_ref)
        segmv_ref[:] = jnp.zeros_like(segmv_ref)
    a1 = a1_ref[:]
    a2 = a2_ref[:]
    we = we_ref[:]
    wd = wd_ref[:]
    bmsg = bmsg_ref[:]
    wvg = wvg_ref[:]
    weu = weu_ref[:]
    H = B // _NSPLIT
    for h in range(_NSPLIT):
        sl = pl.ds(h * H, H)
        oh_s = (scol_ref[sl, :] == _iota_row(N)).astype(_BF)
        oh_t = (tcol_ref[sl, :] == _iota_row(N)).astype(_BF)
        e = e_ref[sl, :]
        rnd = rnd_ref[sl, :]
        d = rnd[:, 3:4]
        pre = (_dot(oh_s, a1) + _dot(oh_t, a2) + e @ we + d * wd + bmsg)
        m = _silu(pre)
        keyf = (scol_ref[sl, :] * N + tcol_ref[sl, :]).astype(jnp.float32)
        enew_ref[sl, :] = jnp.concatenate(
            [e + m @ weu, keyf, jnp.zeros((H, 15), jnp.float32)], axis=1)
        gate = m @ wvg
        mv = jnp.concatenate([rnd[:, 0:1] * gate, rnd[:, 1:2] * gate,
                              rnd[:, 2:3] * gate], axis=1)
        segm_ref[:] += _dot_t(oh_t, m.astype(_BF))
        segmv_ref[:] += _dot_t(oh_t, mv.astype(_BF))


def _node_update_kernel(s_ref, segm_ref, segmv_ref, cnt_ref,
                        wupd_ref, ws1_ref, ws2_ref,
                        snew_ref, vl_ref, a1_ref, a2_ref):
    cnt = jnp.maximum(cnt_ref[:, 0:1], 1.0)
    snew = s_ref[:] + (segm_ref[:] / cnt) @ wupd_ref[:]
    snew_ref[:] = snew
    vl_ref[:] = segmv_ref[:] / cnt
    a1_ref[:] = (snew @ ws1_ref[:]).astype(_BF)
    a2_ref[:] = (snew @ ws2_ref[:]).astype(_BF)


def _final_node_kernel(s_ref, v0_ref, v1_ref, posc_ref, bcol_ref, brow_ref,
                       wsm_ref, bsm_ref, w0f_ref, wcoord_ref,
                       wbond_ref, bbond_ref, b0_ref,
                       z_ref, c16_ref, wb2_ref, c0_ref):
    s2 = _silu(s_ref[:] @ wsm_ref[:] + bsm_ref[:])
    z_ref[:] = (s2 @ w0f_ref[:]).astype(_BF)
    v = v0_ref[:] + v1_ref[:]
    wc = wcoord_ref[:]
    c0c = v[:, 0:VDIM] @ wc
    c1c = v[:, VDIM:2 * VDIM] @ wc
    c2c = v[:, 2 * VDIM:3 * VDIM] @ wc
    zero5 = jnp.zeros((N, 5), jnp.float32)
    coords = posc_ref[:] + jnp.concatenate([c0c, c1c, c2c, zero5], axis=1)
    ohB = (bcol_ref[:] == _iota_row(G)).astype(jnp.float32)
    ohBT = (_iota_col(G) == brow_ref[:]).astype(jnp.float32)
    csum = ohBT @ coords
    cnt = jnp.sum(ohBT, axis=1, keepdims=True)
    mean = csum / jnp.maximum(cnt, 1.0)
    cc = coords - ohB @ mean
    hi = cc.astype(_BF)
    lo = (cc - hi.astype(jnp.float32)).astype(_BF)
    c16_ref[:] = jnp.concatenate([hi, lo], axis=1)
    wb2_ref[:] = wbond_ref[:] @ w0f_ref[:]
    c0_ref[:] = bbond_ref[:] @ w0f_ref[:] + b0_ref[:]


def _final_edge_kernel(icol_ref, jcol_ref, echk_ref, rchk_ref, z_ref,
                       c16_ref, wb2_ref, c0_ref, w0d_ref, w1_ref, b1_ref,
                       out_ref):
    oh_i = (icol_ref[:] == _iota_row(N)).astype(_BF)
    oh_j = (jcol_ref[:] == _iota_row(N)).astype(_BF)
    zp = _dot(oh_i + oh_j, z_ref[:])
    dc16 = _dot(oh_i - oh_j, c16_ref[:])
    dc = dc16[:, 0:8] + dc16[:, 8:16]
    dd = jnp.sum(dc * dc, axis=1, keepdims=True)
    k2f = (icol_ref[:] * N + jcol_ref[:]).astype(jnp.float32)
    ok = (rchk_ref[:, 32:33] == k2f).astype(jnp.float32)
    esym = 0.5 * (echk_ref[:, 0:EDIM] + rchk_ref[:, 0:EDIM] * ok)
    h = _silu(zp + esym @ wb2_ref[:] + dd * w0d_ref[:] + c0_ref[:])
    out_ref[:] = h @ w1_ref[:] + b1_ref[:]


_SC_NC = 2
_SC_NS = 16
_NW = _SC_NC * _SC_NS
_CH = E // _NW
_SUB = 128
_NSUB = _CH // _SUB


def _sc_scatter_ids(k1_2d, ids_2d):
    mesh = plsc.VectorSubcoreMesh(core_axis_name="c", subcore_axis_name="s")

    @functools.partial(
        pl.kernel, mesh=mesh,
        out_type=jax.ShapeDtypeStruct((N * N,), jnp.int32),
        scratch_types=[pltpu.VMEM((_NSUB, _SUB), jnp.int32),
                       pltpu.VMEM((_NSUB, _SUB), jnp.int32),
                       pltpu.SemaphoreType.DMA],
    )
    def k(k1_hbm, ids_hbm, tbl_hbm, kidx_v, vals_v, sem):
        wid = lax.axis_index("s") * _SC_NC + lax.axis_index("c")
        row0 = wid * _NSUB
        pltpu.sync_copy(k1_hbm.at[pl.ds(row0, _NSUB)], kidx_v)
        pltpu.sync_copy(ids_hbm.at[pl.ds(row0, _NSUB)], vals_v)
        copies = [pltpu.async_copy(vals_v.at[i], tbl_hbm.at[kidx_v.at[i]],
                                   sem) for i in range(_NSUB)]
        for c in copies:
            c.wait()

    return k(k1_2d, ids_2d)


def _sc_resolve(tbl, k2_flat):
    mesh = plsc.VectorSubcoreMesh(core_axis_name="c", subcore_axis_name="s")

    @functools.partial(
        pl.kernel, mesh=mesh,
        compiler_params=pltpu.CompilerParams(use_tc_tiling_on_sc=False),
        out_type=jax.ShapeDtypeStruct((E,), jnp.int32),
        scratch_types=[pltpu.VMEM((_CH,), jnp.int32),
                       pltpu.VMEM((_CH,), jnp.int32),
                       pltpu.SemaphoreType.DMA],
    )
    def k(tbl_hbm, k2_hbm, w2_hbm, k2_v, w2_v, sem):
        wid = lax.axis_index("s") * _SC_NC + lax.axis_index("c")
        base = wid * _CH
        pltpu.sync_copy(k2_hbm.at[pl.ds(base, _CH)], k2_v)
        copies = []
        for i in range(_NSUB):
            sl = pl.ds(i * _SUB, _SUB)
            copies.append(pltpu.async_copy(tbl_hbm.at[k2_v.at[sl]],
                                           w2_v.at[sl], sem))
        for c in copies:
            c.wait()
        for j in range(_CH // 16):
            s16 = pl.ds(j * 16, 16)
            w2_v[s16] = w2_v[s16] & (E - 1)
        pltpu.sync_copy(w2_v, w2_hbm.at[pl.ds(base, _CH)])

    return k(tbl, k2_flat)


def _sc_row_gather(e2chk, w2c):
    mesh = plsc.VectorSubcoreMesh(core_axis_name="c", subcore_axis_name="s")

    @functools.partial(
        pl.kernel, mesh=mesh,
        compiler_params=pltpu.CompilerParams(use_tc_tiling_on_sc=False),
        out_type=jax.ShapeDtypeStruct((E, 48), jnp.float32),
        scratch_types=[pltpu.VMEM((_CH,), jnp.int32),
                       pltpu.VMEM((_CH, 48), jnp.float32),
                       pltpu.SemaphoreType.DMA],
    )
    def k(e2_hbm, w2_hbm, out_hbm, w2_v, rows_v, sem):
        wid = lax.axis_index("s") * _SC_NC + lax.axis_index("c")
        base = wid * _CH
        pltpu.sync_copy(w2_hbm.at[pl.ds(base, _CH)], w2_v)
        copies = []
        for i in range(_NSUB):
            sl = pl.ds(i * _SUB, _SUB)
            copies.append(pltpu.async_copy(e2_hbm.at[w2_v.at[sl]],
                                           rows_v.at[pl.ds(i * _SUB, _SUB)],
                                           sem))
        for c in copies:
            c.wait()
        pltpu.sync_copy(rows_v, out_hbm.at[pl.ds(base, _CH)])

    return k(e2chk, w2c)


def _row(v):
    return v.reshape(1, -1)


def kernel(x, t, pos, edge_index_local, edge_index_global, batch,
           batch_edge_global, params):
    p = params
    src = edge_index_global[0].astype(jnp.int32)
    tgt = edge_index_global[1].astype(jnp.int32)
    beg = batch_edge_global.astype(jnp.int32)
    batch = batch.astype(jnp.int32)
    pos8 = jnp.pad(pos, ((0, 0), (0, 5)))

    scol = src.reshape(E, 1)
    tcol = tgt.reshape(E, 1)
    gcol = beg.reshape(E, 1)
    bcol = batch.reshape(N, 1)
    brow = batch.reshape(1, N)

    wmsg0, wmsg1 = p['Wmsg0'], p['Wmsg1']
    ws1_0, ws2_0 = wmsg0[:SDIM], wmsg0[SDIM:2 * SDIM]
    we_0, wd_0 = wmsg0[2 * SDIM:2 * SDIM + EDIM], _row(wmsg0[2 * SDIM + EDIM])
    ws1_1, ws2_1 = wmsg1[:SDIM], wmsg1[SDIM:2 * SDIM]
    we_1, wd_1 = wmsg1[2 * SDIM:2 * SDIM + EDIM], _row(wmsg1[2 * SDIM + EDIM])
    w0f, w0d = p['W0'][:SDIM], _row(p['W0'][SDIM])
    w1p = jnp.pad(p['W1'], ((0, 0), (0, 3)))
    b1p = _row(jnp.pad(p['b1'], (0, 3)))

    f32 = jnp.float32
    full = lambda shape: pl.BlockSpec(shape, lambda i: (0,) * len(shape))
    ecol = pl.BlockSpec((B, 1), lambda i: (i, 0))
    eblk = lambda w: pl.BlockSpec((B, w), lambda i: (i, 0))

    s, posc, ta1, ta2, u = pl.pallas_call(
        _node_prep_kernel,
        out_shape=[jax.ShapeDtypeStruct((N, SDIM), f32),
                   jax.ShapeDtypeStruct((N, 8), f32),
                   jax.ShapeDtypeStruct((N, SDIM + 16), _BF),
                   jax.ShapeDtypeStruct((N, SDIM + 48), _BF),
                   jax.ShapeDtypeStruct((G, EDIM), f32)],
    )(x, t, pos8, bcol, brow, p['Wta'], _row(p['bta']), p['Wtb'],
      _row(p['btb']), p['Wam'], _row(p['bam']), p['Watm'], _row(p['batm']),
      p['Wbm'], _row(p['bbm']), p['Wbtm'], _row(p['bbtm']), ws1_0, ws2_0)

    tlhs = pltpu.CompilerParams(fuse_transposed_lhs_in_matmul=True)
    e1, rnd, cnt8, segm0, segmv0 = pl.pallas_call(
        _layer0_kernel,
        grid=(NB,),
        in_specs=[ecol, ecol, ecol, full((N, SDIM + 48)),
                  full((N, SDIM + 16)), full((G, EDIM)),
                  full((EDIM, SDIM)), full((1, SDIM)), full((1, SDIM)),
                  full((SDIM, VDIM)), full((SDIM, EDIM))],
        out_specs=[eblk(EDIM), eblk(8), full((N, 8)), full((N, SDIM)),
                   full((N, 3 * VDIM))],
        out_shape=[jax.ShapeDtypeStruct((E, EDIM), f32),
                   jax.ShapeDtypeStruct((E, 8), f32),
                   jax.ShapeDtypeStruct((N, 8), f32),
                   jax.ShapeDtypeStruct((N, SDIM), f32),
                   jax.ShapeDtypeStruct((N, 3 * VDIM), f32)],
        compiler_params=tlhs,
    )(scol, tcol, gcol, ta2, ta1, u, we_0, wd_0, _row(p['bmsg0']),
      p['Wvg0'], p['Weu0'])

    layer_call = pl.pallas_call(
        _layer_kernel,
        grid=(NB,),
        in_specs=[ecol, ecol, eblk(EDIM), eblk(8),
                  full((N, SDIM)), full((N, SDIM)), full((EDIM, SDIM)),
                  full((1, SDIM)), full((1, SDIM)), full((SDIM, VDIM)),
                  full((SDIM, EDIM))],
        out_specs=[eblk(48), full((N, SDIM)), full((N, 3 * VDIM))],
        out_shape=[jax.ShapeDtypeStruct((E, 48), f32),
                   jax.ShapeDtypeStruct((N, SDIM), f32),
                   jax.ShapeDtypeStruct((N, 3 * VDIM), f32)],
        compiler_params=tlhs,
    )
    node_update = pl.pallas_call(
        _node_update_kernel,
        out_shape=[jax.ShapeDtypeStruct((N, SDIM), f32),
                   jax.ShapeDtypeStruct((N, 3 * VDIM), f32),
                   jax.ShapeDtypeStruct((N, SDIM), _BF),
                   jax.ShapeDtypeStruct((N, SDIM), _BF)],
    )

    s1, v0, a1b, a2b = node_update(s, segm0, segmv0, cnt8,
                                   p['Wupd0'], ws1_1, ws2_1)
    e2chk, segm1, segmv1 = layer_call(scol, tcol, e1, rnd, a1b, a2b,
                                   we_1, wd_1, _row(p['bmsg1']),
                                   p['Wvg1'], p['Weu1'])
    s2f, v1, _, _ = node_update(s1, segm1, segmv1, cnt8,
                                p['Wupd1'], ws1_1, ws2_1)

    z, c16, wb2, c0v = pl.pallas_call(
        _final_node_kernel,
        out_shape=[jax.ShapeDtypeStruct((N, SDIM), _BF),
                   jax.ShapeDtypeStruct((N, 16), _BF),
                   jax.ShapeDtypeStruct((EDIM, SDIM), f32),
                   jax.ShapeDtypeStruct((1, SDIM), f32)],
    )(s2f, v0, v1, posc, bcol, brow, p['Wsm'], _row(p['bsm']), w0f,
      p['Wcoord'], p['Wbond'], _row(p['bbond']), _row(p['b0']))

    key1 = src * N + tgt
    key2 = tgt * N + src
    ids = jnp.arange(E, dtype=jnp.int32)
    tbl = _sc_scatter_ids(key1.reshape(E // _SUB, _SUB),
                          ids.reshape(E // _SUB, _SUB))
    w2c = _sc_resolve(tbl, key2)
    r2chk = _sc_row_gather(e2chk, w2c)

    outp = pl.pallas_call(
        _final_edge_kernel,
        grid=(NB,),
        in_specs=[ecol, ecol, eblk(48), eblk(48), full((N, SDIM)),
                  full((N, 16)), full((EDIM, SDIM)), full((1, SDIM)),
                  full((1, SDIM)), full((SDIM, 8)), full((1, 8))],
        out_specs=eblk(8),
        out_shape=jax.ShapeDtypeStruct((E, 8), f32),
    )(tcol, scol, e2chk, r2chk, z, c16, wb2, c0v, w0d, w1p, b1p)

    return outp[:, :NBOND]


# speedup vs baseline: 1.8750x; 1.1303x over previous
import functools
import jax
import jax.numpy as jnp
from jax import lax
from jax.experimental import pallas as pl
from jax.experimental.pallas import tpu as pltpu
from jax.experimental.pallas import tpu_sc as plsc

N = 1024
E = 65536
G = 32
F = 16
SDIM = 256
VDIM = 64
EDIM = 32
NBOND = 5
NLAYERS = 2

B = 1024
NB = E // B

_BF = jnp.bfloat16


def _silu(x):
    return x * (1.0 / (1.0 + jnp.exp(-x)))


def _dot(a, b):
    return jax.lax.dot_general(a, b, (((1,), (0,)), ((), ())),
                               preferred_element_type=jnp.float32)


def _dot_t(a, b):
    return jax.lax.dot_general(a, b, (((0,), (0,)), ((), ())),
                               preferred_element_type=jnp.float32)


def _split16(x8):
    hi = x8.astype(_BF)
    lo = (x8 - hi.astype(jnp.float32)).astype(_BF)
    return jnp.concatenate([hi, lo], axis=1)


def _iota_row(n):
    return jax.lax.broadcasted_iota(jnp.int32, (1, n), 1)


def _iota_col(n):
    return jax.lax.broadcasted_iota(jnp.int32, (n, 1), 0)


def _node_prep_kernel(x_ref, t_ref, pos_ref, bcol_ref, brow_ref,
                      wta_ref, bta_ref, wtb_ref, btb_ref,
                      wam_ref, bam_ref, watm_ref, batm_ref,
                      wbm_ref, bbm_ref, wbtm_ref, bbtm_ref,
                      ws1_ref, ws2_ref,
                      s_ref, posc_ref, ta1_ref, ta2_ref, u_ref):
    x = x_ref[:]
    t = t_ref[:]
    ohB = (bcol_ref[:] == _iota_row(G)).astype(jnp.float32)
    ohBT = (_iota_col(G) == brow_ref[:]).astype(jnp.float32)
    ta = t @ wta_ref[:] + bta_ref[:]
    tb = t @ wtb_ref[:] + btb_ref[:]
    s0 = x @ wam_ref[:] + bam_ref[:] + ohB @ ta
    s = s0 @ watm_ref[:] + batm_ref[:]
    pos = pos_ref[:]
    psum = ohBT @ pos
    cnt = jnp.sum(ohBT, axis=1, keepdims=True)
    mean = psum / jnp.maximum(cnt, 1.0)
    posc = pos - ohB @ mean
    s_ref[:] = s
    posc_ref[:] = posc
    xb = x @ wbm_ref[:] + bbm_ref[:]
    q = xb @ wbtm_ref[:]
    hi = posc.astype(_BF)
    lo = (posc - hi.astype(jnp.float32)).astype(_BF)
    u_ref[:] = tb @ wbtm_ref[:] + bbtm_ref[:]
    a1 = (s @ ws1_ref[:]).astype(_BF)
    a2 = (s @ ws2_ref[:]).astype(_BF)
    ta1_ref[:] = jnp.concatenate([a1, hi, lo], axis=1)
    ta2_ref[:] = jnp.concatenate([a2, hi, lo, q.astype(_BF)], axis=1)


def _layer0_kernel(scol_ref, tcol_ref, gcol_ref,
                   ta2_ref, ta1_ref, u_ref,
                   we_ref, wd_ref, bmsg_ref, wvg_ref, weu_ref,
                   e1_ref, rnd_ref, cnt_ref, segm_ref, segmv_ref):
    @pl.when(pl.program_id(0) == 0)
    def _():
        cnt_ref[:] = jnp.zeros_like(cnt_ref)
        segm_ref[:] = jnp.zeros_like(segm_ref)
        segmv_ref[:] = jnp.zeros_like(segmv_ref)
    ta2 = ta2_ref[:]
    ta1 = ta1_ref[:]
    u = u_ref[:]
    we = we_ref[:]
    wd = wd_ref[:]
    bmsg = bmsg_ref[:]
    wvg = wvg_ref[:]
    weu = weu_ref[:]
    col3 = (_iota_row(8) == 3).astype(jnp.float32)
    H = B // _NSPLIT
    for h in range(_NSPLIT):
        sl = pl.ds(h * H, H)
        oh_t = (tcol_ref[sl, :] == _iota_row(N)).astype(_BF)
        oh_s = (scol_ref[sl, :] == _iota_row(N)).astype(_BF)
        oh_g = (gcol_ref[sl, :] == _iota_row(G)).astype(jnp.float32)
        at = _dot(oh_t, ta2)
        asrc = _dot(oh_s, ta1)
        pos_t = at[:, SDIM:SDIM + 8] + at[:, SDIM + 8:SDIM + 16]
        pos_s = asrc[:, SDIM:SDIM + 8] + asrc[:, SDIM + 8:SDIM + 16]
        e0 = at[:, SDIM + 16:SDIM + 48] + oh_g @ u
        r = pos_t - pos_s
        d2 = jnp.sum(r * r, axis=1, keepdims=True)
        d = jnp.sqrt(jnp.maximum(d2, 1e-6))
        rn = r / (1.0 + d)
        rnd_ref[sl, :] = rn + d * col3
        pre = (asrc[:, 0:SDIM] + at[:, 0:SDIM] + e0 @ we
               + d * wd + bmsg)
        m = _silu(pre)
        e1_ref[sl, :] = e0 + m @ weu
        gate = m @ wvg
        mv = jnp.concatenate([rn[:, 0:1] * gate, rn[:, 1:2] * gate,
                              rn[:, 2:3] * gate], axis=1)
        cnt_ref[:] += _dot_t(oh_t, jnp.ones((H, 8), _BF))
        segm_ref[:] += _dot_t(oh_t, m.astype(_BF))
        segmv_ref[:] += _dot_t(oh_t, mv.astype(_BF))


_NSPLIT = 2


def _layer_kernel(scol_ref, tcol_ref, e_ref, rnd_ref,
                  a1_ref, a2_ref, we_ref, wd_ref, bmsg_ref,
                  wvg_ref, weu_ref,
                  enew_ref, segm_ref, segmv_ref):
    @pl.when(pl.program_id(0) == 0)
    def _():
        segm_ref[:] = jnp.zeros_like(segm_ref)
        segmv_ref[:] = jnp.zeros_like(segmv_ref)
    a1 = a1_ref[:]
    a2 = a2_ref[:]
    we = we_ref[:]
    wd = wd_ref[:]
    bmsg = bmsg_ref[:]
    wvg = wvg_ref[:]
    weu = weu_ref[:]
    H = B // _NSPLIT
    for h in range(_NSPLIT):
        sl = pl.ds(h * H, H)
        oh_s = (scol_ref[sl, :] == _iota_row(N)).astype(_BF)
        oh_t = (tcol_ref[sl, :] == _iota_row(N)).astype(_BF)
        e = e_ref[sl, :]
        rnd = rnd_ref[sl, :]
        d = rnd[:, 3:4]
        pre = (_dot(oh_s, a1) + _dot(oh_t, a2) + e @ we + d * wd + bmsg)
        m = _silu(pre)
        keyf = (scol_ref[sl, :] * N + tcol_ref[sl, :]).astype(jnp.float32)
        enew_ref[sl, :] = jnp.concatenate(
            [e + m @ weu, keyf, jnp.zeros((H, 15), jnp.float32)], axis=1)
        gate = m @ wvg
        mv = jnp.concatenate([rnd[:, 0:1] * gate, rnd[:, 1:2] * gate,
                              rnd[:, 2:3] * gate], axis=1)
        segm_ref[:] += _dot_t(oh_t, m.astype(_BF))
        segmv_ref[:] += _dot_t(oh_t, mv.astype(_BF))


def _node_update_kernel(s_ref, segm_ref, segmv_ref, cnt_ref,
                        wupd_ref, ws1_ref, ws2_ref,
                        snew_ref, vl_ref, a1_ref, a2_ref):
    cnt = jnp.maximum(cnt_ref[:, 0:1], 1.0)
    snew = s_ref[:] + (segm_ref[:] / cnt) @ wupd_ref[:]
    snew_ref[:] = snew
    vl_ref[:] = segmv_ref[:] / cnt
    a1_ref[:] = (snew @ ws1_ref[:]).astype(_BF)
    a2_ref[:] = (snew @ ws2_ref[:]).astype(_BF)


def _final_node_kernel(s_ref, v0_ref, v1_ref, posc_ref, bcol_ref, brow_ref,
                       wsm_ref, bsm_ref, w0f_ref, wcoord_ref,
                       wbond_ref, bbond_ref, b0_ref,
                       z_ref, c16_ref, wb2_ref, c0_ref):
    s2 = _silu(s_ref[:] @ wsm_ref[:] + bsm_ref[:])
    z_ref[:] = (s2 @ w0f_ref[:]).astype(_BF)
    v = v0_ref[:] + v1_ref[:]
    wc = wcoord_ref[:]
    c0c = v[:, 0:VDIM] @ wc
    c1c = v[:, VDIM:2 * VDIM] @ wc
    c2c = v[:, 2 * VDIM:3 * VDIM] @ wc
    zero5 = jnp.zeros((N, 5), jnp.float32)
    coords = posc_ref[:] + jnp.concatenate([c0c, c1c, c2c, zero5], axis=1)
    ohB = (bcol_ref[:] == _iota_row(G)).astype(jnp.float32)
    ohBT = (_iota_col(G) == brow_ref[:]).astype(jnp.float32)
    csum = ohBT @ coords
    cnt = jnp.sum(ohBT, axis=1, keepdims=True)
    mean = csum / jnp.maximum(cnt, 1.0)
    cc = coords - ohB @ mean
    hi = cc.astype(_BF)
    lo = (cc - hi.astype(jnp.float32)).astype(_BF)
    c16_ref[:] = jnp.concatenate([hi, lo], axis=1)
    wb2_ref[:] = wbond_ref[:] @ w0f_ref[:]
    c0_ref[:] = bbond_ref[:] @ w0f_ref[:] + b0_ref[:]


def _final_edge_kernel(icol_ref, jcol_ref, echk_ref, rchk_ref, z_ref,
                       c16_ref, wb2_ref, c0_ref, w0d_ref, w1_ref, b1_ref,
                       out_ref):
    oh_i = (icol_ref[:] == _iota_row(N)).astype(_BF)
    oh_j = (jcol_ref[:] == _iota_row(N)).astype(_BF)
    zp = _dot(oh_i + oh_j, z_ref[:])
    dc16 = _dot(oh_i - oh_j, c16_ref[:])
    dc = dc16[:, 0:8] + dc16[:, 8:16]
    dd = jnp.sum(dc * dc, axis=1, keepdims=True)
    k2f = (icol_ref[:] * N + jcol_ref[:]).astype(jnp.float32)
    ok = (rchk_ref[:, 32:33] == k2f).astype(jnp.float32)
    esym = 0.5 * (echk_ref[:, 0:EDIM] + rchk_ref[:, 0:EDIM] * ok)
    h = _silu(zp + esym @ wb2_ref[:] + dd * w0d_ref[:] + c0_ref[:])
    out_ref[:] = h @ w1_ref[:] + b1_ref[:]


_SC_NC = 2
_SC_NS = 16
_NW = _SC_NC * _SC_NS
_CH = E // _NW
_SUB = 128
_NSUB = _CH // _SUB


def _sc_scatter_ids(k1_2d, ids_2d):
    mesh = plsc.VectorSubcoreMesh(core_axis_name="c", subcore_axis_name="s")

    @functools.partial(
        pl.kernel, mesh=mesh,
        out_type=jax.ShapeDtypeStruct((N * N,), jnp.int32),
        scratch_types=[pltpu.VMEM((_NSUB, _SUB), jnp.int32),
                       pltpu.VMEM((_NSUB, _SUB), jnp.int32),
                       pltpu.SemaphoreType.DMA],
    )
    def k(k1_hbm, ids_hbm, tbl_hbm, kidx_v, vals_v, sem):
        wid = lax.axis_index("s") * _SC_NC + lax.axis_index("c")
        row0 = wid * _NSUB
        pltpu.sync_copy(k1_hbm.at[pl.ds(row0, _NSUB)], kidx_v)
        pltpu.sync_copy(ids_hbm.at[pl.ds(row0, _NSUB)], vals_v)
        copies = [pltpu.async_copy(vals_v.at[i], tbl_hbm.at[kidx_v.at[i]],
                                   sem) for i in range(_NSUB)]
        for c in copies:
            c.wait()

    return k(k1_2d, ids_2d)


def _sc_resolve(tbl, k2_flat):
    mesh = plsc.VectorSubcoreMesh(core_axis_name="c", subcore_axis_name="s")

    @functools.partial(
        pl.kernel, mesh=mesh,
        compiler_params=pltpu.CompilerParams(use_tc_tiling_on_sc=False),
        out_type=jax.ShapeDtypeStruct((E,), jnp.int32),
        scratch_types=[pltpu.VMEM((_CH,), jnp.int32),
                       pltpu.VMEM((_CH,), jnp.int32),
                       pltpu.SemaphoreType.DMA],
    )
    def k(tbl_hbm, k2_hbm, w2_hbm, k2_v, w2_v, sem):
        wid = lax.axis_index("s") * _SC_NC + lax.axis_index("c")
        base = wid * _CH
        pltpu.sync_copy(k2_hbm.at[pl.ds(base, _CH)], k2_v)
        copies = []
        for i in range(_NSUB):
            sl = pl.ds(i * _SUB, _SUB)
            copies.append(pltpu.async_copy(tbl_hbm.at[k2_v.at[sl]],
                                           w2_v.at[sl], sem))
        for c in copies:
            c.wait()
        for j in range(_CH // 16):
            s16 = pl.ds(j * 16, 16)
            w2_v[s16] = w2_v[s16] & (E - 1)
        pltpu.sync_copy(w2_v, w2_hbm.at[pl.ds(base, _CH)])

    return k(tbl, k2_flat)


def _sc_row_gather(e2chk, w2c):
    mesh = plsc.VectorSubcoreMesh(core_axis_name="c", subcore_axis_name="s")

    @functools.partial(
        pl.kernel, mesh=mesh,
        compiler_params=pltpu.CompilerParams(use_tc_tiling_on_sc=False),
        out_type=jax.ShapeDtypeStruct((E, 48), jnp.float32),
        scratch_types=[pltpu.VMEM((_CH,), jnp.int32),
                       pltpu.VMEM((_CH, 48), jnp.float32),
                       pltpu.SemaphoreType.DMA],
    )
    def k(e2_hbm, w2_hbm, out_hbm, w2_v, rows_v, sem):
        wid = lax.axis_index("s") * _SC_NC + lax.axis_index("c")
        base = wid * _CH
        pltpu.sync_copy(w2_hbm.at[pl.ds(base, _CH)], w2_v)
        copies = []
        for i in range(_NSUB):
            sl = pl.ds(i * _SUB, _SUB)
            copies.append(pltpu.async_copy(e2_hbm.at[w2_v.at[sl]],
                                           rows_v.at[pl.ds(i * _SUB, _SUB)],
                                           sem))
        for c in copies:
            c.wait()
        pltpu.sync_copy(rows_v, out_hbm.at[pl.ds(base, _CH)])

    return k(e2chk, w2c)


def _row(v):
    return v.reshape(1, -1)


def kernel(x, t, pos, edge_index_local, edge_index_global, batch,
           batch_edge_global, params):
    p = params
    src = edge_index_global[0].astype(jnp.int32)
    tgt = edge_index_global[1].astype(jnp.int32)
    beg = batch_edge_global.astype(jnp.int32)
    batch = batch.astype(jnp.int32)
    pos8 = jnp.pad(pos, ((0, 0), (0, 5)))

    scol = src.reshape(E, 1)
    tcol = tgt.reshape(E, 1)
    gcol = beg.reshape(E, 1)
    bcol = batch.reshape(N, 1)
    brow = batch.reshape(1, N)

    wmsg0, wmsg1 = p['Wmsg0'], p['Wmsg1']
    ws1_0, ws2_0 = wmsg0[:SDIM], wmsg0[SDIM:2 * SDIM]
    we_0, wd_0 = wmsg0[2 * SDIM:2 * SDIM + EDIM], _row(wmsg0[2 * SDIM + EDIM])
    ws1_1, ws2_1 = wmsg1[:SDIM], wmsg1[SDIM:2 * SDIM]
    we_1, wd_1 = wmsg1[2 * SDIM:2 * SDIM + EDIM], _row(wmsg1[2 * SDIM + EDIM])
    w0f, w0d = p['W0'][:SDIM], _row(p['W0'][SDIM])
    w1p = jnp.pad(p['W1'], ((0, 0), (0, 3)))
    b1p = _row(jnp.pad(p['b1'], (0, 3)))

    f32 = jnp.float32
    full = lambda shape: pl.BlockSpec(shape, lambda i: (0,) * len(shape))
    ecol = pl.BlockSpec((B, 1), lambda i: (i, 0))
    eblk = lambda w: pl.BlockSpec((B, w), lambda i: (i, 0))

    s, posc, ta1, ta2, u = pl.pallas_call(
        _node_prep_kernel,
        out_shape=[jax.ShapeDtypeStruct((N, SDIM), f32),
                   jax.ShapeDtypeStruct((N, 8), f32),
                   jax.ShapeDtypeStruct((N, SDIM + 16), _BF),
                   jax.ShapeDtypeStruct((N, SDIM + 48), _BF),
                   jax.ShapeDtypeStruct((G, EDIM), f32)],
    )(x, t, pos8, bcol, brow, p['Wta'], _row(p['bta']), p['Wtb'],
      _row(p['btb']), p['Wam'], _row(p['bam']), p['Watm'], _row(p['batm']),
      p['Wbm'], _row(p['bbm']), p['Wbtm'], _row(p['bbtm']), ws1_0, ws2_0)

    e1, rnd, cnt8, segm0, segmv0 = pl.pallas_call(
        _layer0_kernel,
        grid=(NB,),
        in_specs=[ecol, ecol, ecol, full((N, SDIM + 48)),
                  full((N, SDIM + 16)), full((G, EDIM)),
                  full((EDIM, SDIM)), full((1, SDIM)), full((1, SDIM)),
                  full((SDIM, VDIM)), full((SDIM, EDIM))],
        out_specs=[eblk(EDIM), eblk(8), full((N, 8)), full((N, SDIM)),
                   full((N, 3 * VDIM))],
        out_shape=[jax.ShapeDtypeStruct((E, EDIM), f32),
                   jax.ShapeDtypeStruct((E, 8), f32),
                   jax.ShapeDtypeStruct((N, 8), f32),
                   jax.ShapeDtypeStruct((N, SDIM), f32),
                   jax.ShapeDtypeStruct((N, 3 * VDIM), f32)],
    )(scol, tcol, gcol, ta2, ta1, u, we_0, wd_0, _row(p['bmsg0']),
      p['Wvg0'], p['Weu0'])

    layer_call = pl.pallas_call(
        _layer_kernel,
        grid=(NB,),
        in_specs=[ecol, ecol, eblk(EDIM), eblk(8),
                  full((N, SDIM)), full((N, SDIM)), full((EDIM, SDIM)),
                  full((1, SDIM)), full((1, SDIM)), full((SDIM, VDIM)),
                  full((SDIM, EDIM))],
        out_specs=[eblk(48), full((N, SDIM)), full((N, 3 * VDIM))],
        out_shape=[jax.ShapeDtypeStruct((E, 48), f32),
                   jax.ShapeDtypeStruct((N, SDIM), f32),
                   jax.ShapeDtypeStruct((N, 3 * VDIM), f32)],
    )
    node_update = pl.pallas_call(
        _node_update_kernel,
        out_shape=[jax.ShapeDtypeStruct((N, SDIM), f32),
                   jax.ShapeDtypeStruct((N, 3 * VDIM), f32),
                   jax.ShapeDtypeStruct((N, SDIM), _BF),
                   jax.ShapeDtypeStruct((N, SDIM), _BF)],
    )

    s1, v0, a1b, a2b = node_update(s, segm0, segmv0, cnt8,
                                   p['Wupd0'], ws1_1, ws2_1)
    e2chk, segm1, segmv1 = layer_call(scol, tcol, e1, rnd, a1b, a2b,
                                   we_1, wd_1, _row(p['bmsg1']),
                                   p['Wvg1'], p['Weu1'])
    s2f, v1, _, _ = node_update(s1, segm1, segmv1, cnt8,
                                p['Wupd1'], ws1_1, ws2_1)

    z, c16, wb2, c0v = pl.pallas_call(
        _final_node_kernel,
        out_shape=[jax.ShapeDtypeStruct((N, SDIM), _BF),
                   jax.ShapeDtypeStruct((N, 16), _BF),
                   jax.ShapeDtypeStruct((EDIM, SDIM), f32),
                   jax.ShapeDtypeStruct((1, SDIM), f32)],
    )(s2f, v0, v1, posc, bcol, brow, p['Wsm'], _row(p['bsm']), w0f,
      p['Wcoord'], p['Wbond'], _row(p['bbond']), _row(p['b0']))

    key1 = src * N + tgt
    key2 = tgt * N + src
    ids = jnp.arange(E, dtype=jnp.int32)
    tbl = _sc_scatter_ids(key1.reshape(E // _SUB, _SUB),
                          ids.reshape(E // _SUB, _SUB))
    w2c = _sc_resolve(tbl, key2)
    r2chk = _sc_row_gather(e2chk, w2c)

    outp = pl.pallas_call(
        _final_edge_kernel,
        grid=(NB,),
        in_specs=[ecol, ecol, eblk(48), eblk(48), full((N, SDIM)),
                  full((N, 16)), full((EDIM, SDIM)), full((1, SDIM)),
                  full((1, SDIM)), full((SDIM, 8)), full((1, 8))],
        out_specs=eblk(8),
        out_shape=jax.ShapeDtypeStruct((E, 8), f32),
    )(tcol, scol, e2chk, r2chk, z, c16, wb2, c0v, w0d, w1p, b1p)

    return outp[:, :NBOND]
